# Initial kernel scaffold; baseline (speedup 1.0000x reference)
import jax
import jax.numpy as jnp
from jax.experimental import pallas as pl


def kernel(h, edge_index, edge_attr, eW1, eb1, eW2, eb2, nW1, nb1, nW2, nb2):
    raise NotImplementedError("write your pallas kernel here")


# speedup vs baseline: 2.9095x; 2.9095x over previous
import functools

import jax
import jax.numpy as jnp
from jax import lax
from jax.experimental import pallas as pl
from jax.experimental.pallas import tpu as pltpu
from jax.experimental.pallas import tpu_sc as plsc

_N = 10000
_E = 320000
_D = 128
_DE = 16
_H = 128

_NC = 2
_NS = 16
_NW = _NC * _NS

_PER_W = _E // _NW
_CHUNK = 400
_NCHUNK = _PER_W // _CHUNK
_BATCH = 80
_KB = _CHUNK // _BATCH
_NHALF = _N // _NC
_ACC_PAD = 5120
_ACC_ROWS = _ACC_PAD + 8
_DUMP = _ACC_PAD
_ZROWS = _ACC_PAD // _NS
_SC_PER_TILE = _E // _NS
_SC_NCHUNK = _SC_PER_TILE // _CHUNK

_BN = 2000


def _silu(x):
    return x / (1.0 + jnp.exp(-x))


def _premix_body(h_ref, wa_ref, wb_ref, t_ref):
    hb = h_ref[...]
    t_ref[0] = jnp.dot(hb, wa_ref[...], preferred_element_type=jnp.float32)
    t_ref[1] = jnp.dot(hb, wb_ref[...], preferred_element_type=jnp.float32)


def _premix(h, wa, wb):
    return pl.pallas_call(
        _premix_body,
        grid=(_N // _BN,),
        in_specs=[
            pl.BlockSpec((_BN, _D), lambda n: (n, 0)),
            pl.BlockSpec((_D, _H), lambda n: (0, 0)),
            pl.BlockSpec((_D, _H), lambda n: (0, 0)),
        ],
        out_specs=pl.BlockSpec((2, _BN, _H), lambda n: (0, n, 0)),
        out_shape=jax.ShapeDtypeStruct((2, _N, _H), jnp.float32),
    )(h, wa, wb)


_sc_mesh = plsc.VectorSubcoreMesh(
    core_axis_name="c", subcore_axis_name="s", num_cores=_NC, num_subcores=_NS
)


@functools.partial(
    pl.kernel,
    out_type=jax.ShapeDtypeStruct((2, _E, _H), jnp.float32),
    mesh=_sc_mesh,
    scratch_types=[
        pltpu.VMEM((_CHUNK,), jnp.int32),
        pltpu.VMEM((_CHUNK, _H), jnp.float32),
        pltpu.SemaphoreType.DMA,
    ],
)
def _sc_gather(t0_hbm, t1_hbm, row_hbm, col_hbm, g_hbm, idx_v, rows_v, sem):
    c = lax.axis_index("c")
    s = lax.axis_index("s")
    wid = c * _NS + s
    base0 = wid * _PER_W

    def one_half(half, tab_hbm, idx_hbm, ci):
        base = base0 + ci * _CHUNK
        pltpu.sync_copy(idx_hbm.at[pl.ds(base, _CHUNK)], idx_v)
        cps = [
            pltpu.async_copy(
                tab_hbm.at[idx_v.at[pl.ds(j * _BATCH, _BATCH)]],
                rows_v.at[pl.ds(j * _BATCH, _BATCH)],
                sem,
            )
            for j in range(_KB)
        ]
        for cp in cps:
            cp.wait()
        pltpu.sync_copy(rows_v, g_hbm.at[half, pl.ds(base, _CHUNK)])

    def chunk_body(ci, carry):
        one_half(0, t0_hbm, row_hbm, ci)
        one_half(1, t1_hbm, col_hbm, ci)
        return carry

    lax.fori_loop(0, _NCHUNK, chunk_body, 0)


def _edge_body(g_ref, attr_ref, w1c_ref, b1_ref, w2_ref, b2_ref, f_ref):
    z = (
        g_ref[0]
        + g_ref[1]
        + jnp.dot(attr_ref[...], w1c_ref[...], preferred_element_type=jnp.float32)
        + b1_ref[...]
    )
    z = _silu(z)
    f = jnp.dot(z, w2_ref[...], preferred_element_type=jnp.float32) + b2_ref[...]
    f_ref[...] = _silu(f)


def _edge_mlp(g, attr, w1c, b1, w2, b2):
    return pl.pallas_call(
        _edge_body,
        grid=(_E // _BN,),
        in_specs=[
            pl.BlockSpec((2, _BN, _H), lambda n: (0, n, 0)),
            pl.BlockSpec((_BN, _DE), lambda n: (n, 0)),
            pl.BlockSpec((_DE, _H), lambda n: (0, 0)),
            pl.BlockSpec((1, _H), lambda n: (0, 0)),
            pl.BlockSpec((_H, _H), lambda n: (0, 0)),
            pl.BlockSpec((1, _H), lambda n: (0, 0)),
        ],
        out_specs=pl.BlockSpec((_BN, _H), lambda n: (n, 0)),
        out_shape=jax.ShapeDtypeStruct((_E, _H), jnp.float32),
    )(g, attr, w1c, b1, w2, b2)


@functools.partial(
    pl.kernel,
    out_type=jax.ShapeDtypeStruct((_N, _H), jnp.float32),
    mesh=_sc_mesh,
    scratch_types=[
        pltpu.VMEM((_BATCH,), jnp.int32),
        pltpu.VMEM((_BATCH,), jnp.int32),
        pltpu.VMEM((_CHUNK, _H), jnp.float32),
        pltpu.VMEM_SHARED((_ACC_ROWS, _H), jnp.float32),
    ],
)
def _sc_scatter(f_hbm, row_hbm, out_hbm, idx_v, idx2_v, fv, acc):
    c = lax.axis_index("c")
    s = lax.axis_index("s")
    lo = c * _NHALF

    def zero_row(i, carry):
        for j in range(_H // 16):
            fv[i, pl.ds(j * 16, 16)] = jnp.zeros((16,), jnp.float32)
        return carry

    lax.fori_loop(0, _ZROWS, zero_row, 0)
    pltpu.sync_copy(fv.at[pl.ds(0, _ZROWS)], acc.at[pl.ds(s * _ZROWS, _ZROWS)])
    plsc.subcore_barrier()

    def chunk_body(ci, carry):
        eb = s * _SC_PER_TILE + ci * _CHUNK
        pltpu.sync_copy(f_hbm.at[pl.ds(eb, _CHUNK)], fv)
        for j in range(_KB):
            pltpu.sync_copy(row_hbm.at[pl.ds(eb + j * _BATCH, _BATCH)], idx_v)
            for k in range(_BATCH // 16):
                v = idx_v[pl.ds(k * 16, 16)] - lo
                ok = (v >= 0) & (v < _NHALF)
                idx2_v[pl.ds(k * 16, 16)] = jnp.where(ok, v, _DUMP)
            pltpu.sync_copy(
                fv.at[pl.ds(j * _BATCH, _BATCH)], acc.at[idx2_v], add=True
            )
        return carry

    lax.fori_loop(0, _SC_NCHUNK, chunk_body, 0)
    plsc.subcore_barrier()
    @pl.when(s < _NS - 1)
    def _():
        pltpu.sync_copy(
            acc.at[pl.ds(s * _ZROWS, _ZROWS)],
            out_hbm.at[pl.ds(lo + s * _ZROWS, _ZROWS)],
        )

    @pl.when(s == _NS - 1)
    def _():
        pltpu.sync_copy(
            acc.at[pl.ds((_NS - 1) * _ZROWS, _NHALF - (_NS - 1) * _ZROWS)],
            out_hbm.at[pl.ds(lo + (_NS - 1) * _ZROWS, _NHALF - (_NS - 1) * _ZROWS)],
        )


def _node_body(h_ref, p_ref, w1a_ref, w1b_ref, b1_ref, w2_ref, b2_ref, o_ref):
    hb = h_ref[...]
    agg = p_ref[...]
    z = _silu(
        jnp.dot(hb, w1a_ref[...], preferred_element_type=jnp.float32)
        + jnp.dot(agg, w1b_ref[...], preferred_element_type=jnp.float32)
        + b1_ref[...]
    )
    o_ref[...] = (
        jnp.dot(z, w2_ref[...], preferred_element_type=jnp.float32) + b2_ref[...] + hb
    )


def _node_mlp(h, parts, w1a, w1b, b1, w2, b2):
    return pl.pallas_call(
        _node_body,
        grid=(_N // _BN,),
        in_specs=[
            pl.BlockSpec((_BN, _D), lambda n: (n, 0)),
            pl.BlockSpec((_BN, _H), lambda n: (n, 0)),
            pl.BlockSpec((_D, _H), lambda n: (0, 0)),
            pl.BlockSpec((_H, _H), lambda n: (0, 0)),
            pl.BlockSpec((1, _H), lambda n: (0, 0)),
            pl.BlockSpec((_H, _D), lambda n: (0, 0)),
            pl.BlockSpec((1, _D), lambda n: (0, 0)),
        ],
        out_specs=pl.BlockSpec((_BN, _D), lambda n: (n, 0)),
        out_shape=jax.ShapeDtypeStruct((_N, _D), jnp.float32),
    )(h, parts, w1a, w1b, b1, w2, b2)


def kernel(h, edge_index, edge_attr, eW1, eb1, eW2, eb2, nW1, nb1, nW2, nb2):
    row = edge_index[0]
    col = edge_index[1]
    w1a = eW1[:_D]
    w1b = eW1[_D : 2 * _D]
    w1c = eW1[2 * _D :]

    t = _premix(h, w1a, w1b)
    g = _sc_gather(t[0], t[1], row, col)
    f = _edge_mlp(
        g, edge_attr, w1c, eb1.reshape(1, _H), eW2, eb2.reshape(1, _H)
    )
    parts = _sc_scatter(f, row)
    out = _node_mlp(
        h,
        parts,
        nW1[:_D],
        nW1[_D:],
        nb1.reshape(1, _H),
        nW2,
        nb2.reshape(1, _D),
    )
    return (out, edge_attr)


# speedup vs baseline: 3.9129x; 1.3449x over previous
import functools

import jax
import jax.numpy as jnp
from jax import lax
from jax.experimental import pallas as pl
from jax.experimental.pallas import tpu as pltpu
from jax.experimental.pallas import tpu_sc as plsc

_N = 10000
_E = 320000
_D = 128
_DE = 16
_H = 128

_NC = 2
_NS = 16
_NW = _NC * _NS

_PER_W = _E // _NW
_CHUNK = 400
_NCHUNK = _PER_W // _CHUNK
_BATCH = 80
_KB = _CHUNK // _BATCH
_ACC_PAD = 10240
_ZTILE = _ACC_PAD // _NS
_NBATCH = _PER_W // _BATCH

_BN = 2000


def _silu(x):
    return x / (1.0 + jnp.exp(-x))


def _premix_body(h_ref, wa_ref, wb_ref, t_ref):
    hb = h_ref[...]
    t_ref[0] = jnp.dot(hb, wa_ref[...], preferred_element_type=jnp.float32)
    t_ref[1] = jnp.dot(hb, wb_ref[...], preferred_element_type=jnp.float32)


def _premix(h, wa, wb):
    return pl.pallas_call(
        _premix_body,
        grid=(_N // _BN,),
        in_specs=[
            pl.BlockSpec((_BN, _D), lambda n: (n, 0)),
            pl.BlockSpec((_D, _H), lambda n: (0, 0)),
            pl.BlockSpec((_D, _H), lambda n: (0, 0)),
        ],
        out_specs=pl.BlockSpec((2, _BN, _H), lambda n: (0, n, 0)),
        out_shape=jax.ShapeDtypeStruct((2, _N, _H), jnp.float32),
    )(h, wa, wb)


_sc_mesh = plsc.VectorSubcoreMesh(
    core_axis_name="c", subcore_axis_name="s", num_cores=_NC, num_subcores=_NS
)


@functools.partial(
    pl.kernel,
    out_type=jax.ShapeDtypeStruct((2, _E, _H), jnp.float32),
    mesh=_sc_mesh,
    scratch_types=[
        pltpu.VMEM((_CHUNK,), jnp.int32),
        pltpu.VMEM((_CHUNK, _H), jnp.float32),
        pltpu.SemaphoreType.DMA,
    ],
)
def _sc_gather(t0_hbm, t1_hbm, row_hbm, col_hbm, g_hbm, idx_v, rows_v, sem):
    c = lax.axis_index("c")
    s = lax.axis_index("s")
    wid = c * _NS + s
    base0 = wid * _PER_W

    def one_half(half, tab_hbm, idx_hbm, ci):
        base = base0 + ci * _CHUNK
        pltpu.sync_copy(idx_hbm.at[pl.ds(base, _CHUNK)], idx_v)
        cps = [
            pltpu.async_copy(
                tab_hbm.at[idx_v.at[pl.ds(j * _BATCH, _BATCH)]],
                rows_v.at[pl.ds(j * _BATCH, _BATCH)],
                sem,
            )
            for j in range(_KB)
        ]
        for cp in cps:
            cp.wait()
        pltpu.sync_copy(rows_v, g_hbm.at[half, pl.ds(base, _CHUNK)])

    def chunk_body(ci, carry):
        one_half(0, t0_hbm, row_hbm, ci)
        one_half(1, t1_hbm, col_hbm, ci)
        return carry

    lax.fori_loop(0, _NCHUNK, chunk_body, 0)


def _edge_body(g_ref, attr_ref, w1c_ref, b1_ref, w2_ref, b2_ref, f_ref):
    z = (
        g_ref[0]
        + g_ref[1]
        + jnp.dot(attr_ref[...], w1c_ref[...], preferred_element_type=jnp.float32)
        + b1_ref[...]
    )
    z = _silu(z)
    f = jnp.dot(z, w2_ref[...], preferred_element_type=jnp.float32) + b2_ref[...]
    f_ref[...] = _silu(f)


def _edge_mlp(g, attr, w1c, b1, w2, b2):
    return pl.pallas_call(
        _edge_body,
        grid=(_E // _BN,),
        in_specs=[
            pl.BlockSpec((2, _BN, _H), lambda n: (0, n, 0)),
            pl.BlockSpec((_BN, _DE), lambda n: (n, 0)),
            pl.BlockSpec((_DE, _H), lambda n: (0, 0)),
            pl.BlockSpec((1, _H), lambda n: (0, 0)),
            pl.BlockSpec((_H, _H), lambda n: (0, 0)),
            pl.BlockSpec((1, _H), lambda n: (0, 0)),
        ],
        out_specs=pl.BlockSpec((_BN, _H), lambda n: (n, 0)),
        out_shape=jax.ShapeDtypeStruct((_E, _H), jnp.float32),
    )(g, attr, w1c, b1, w2, b2)


@functools.partial(
    pl.kernel,
    out_type=jax.ShapeDtypeStruct((_NC, _N, _H), jnp.float32),
    mesh=_sc_mesh,
    scratch_types=[
        pltpu.VMEM((_BATCH,), jnp.int32),
        pltpu.VMEM((_BATCH,), jnp.int32),
        pltpu.VMEM((_BATCH, _H), jnp.float32),
        pltpu.VMEM((_BATCH, _H), jnp.float32),
        pltpu.VMEM_SHARED((_ACC_PAD, _H), jnp.float32),
        pltpu.SemaphoreType.DMA,
        pltpu.SemaphoreType.DMA,
        pltpu.SemaphoreType.DMA,
        pltpu.SemaphoreType.DMA,
    ],
)
def _sc_scatter(f_hbm, row_hbm, out_hbm, i0, i1, f0, f1, acc, si0, si1, sf0, sf1):
    c = lax.axis_index("c")
    s = lax.axis_index("s")
    wid = c * _NS + s
    base = wid * _PER_W

    def zero_row(i, carry):
        for j in range(_H // 16):
            f0[i, pl.ds(j * 16, 16)] = jnp.zeros((16,), jnp.float32)
        return carry

    lax.fori_loop(0, _BATCH, zero_row, 0)
    for t in range(_ZTILE // _BATCH):
        pltpu.sync_copy(f0, acc.at[pl.ds(s * _ZTILE + t * _BATCH, _BATCH)])
    plsc.subcore_barrier()

    def start(e, iv, fvv, sem_i, sem_f):
        eb = base + e * _BATCH
        pltpu.async_copy(row_hbm.at[pl.ds(eb, _BATCH)], iv, sem_i)
        pltpu.async_copy(f_hbm.at[pl.ds(eb, _BATCH)], fvv, sem_f)

    def drain(e, iv, fvv, sem_i, sem_f):
        eb = base + e * _BATCH
        pltpu.make_async_copy(row_hbm.at[pl.ds(eb, _BATCH)], iv, sem_i).wait()
        pltpu.make_async_copy(f_hbm.at[pl.ds(eb, _BATCH)], fvv, sem_f).wait()

    start(0, i0, f0, si0, sf0)
    start(1, i1, f1, si1, sf1)

    def body(i, carry):
        e = 2 * i
        drain(e, i0, f0, si0, sf0)
        pltpu.sync_copy(f0, acc.at[i0], add=True)
        start(e + 2, i0, f0, si0, sf0)
        drain(e + 1, i1, f1, si1, sf1)
        pltpu.sync_copy(f1, acc.at[i1], add=True)

        @pl.when(i < _NBATCH // 2 - 1)
        def _():
            start(e + 3, i1, f1, si1, sf1)

        return carry

    lax.fori_loop(0, _NBATCH // 2, body, 0)
    drain(_NBATCH - 1, i0, f0, si0, sf0)
    pltpu.sync_copy(f0, acc.at[i0], add=True)
    plsc.subcore_barrier()
    @pl.when(s < _NS - 1)
    def _():
        pltpu.sync_copy(
            acc.at[pl.ds(s * _ZTILE, _ZTILE)],
            out_hbm.at[c, pl.ds(s * _ZTILE, _ZTILE)],
        )

    @pl.when(s == _NS - 1)
    def _():
        pltpu.sync_copy(
            acc.at[pl.ds((_NS - 1) * _ZTILE, _N - (_NS - 1) * _ZTILE)],
            out_hbm.at[c, pl.ds((_NS - 1) * _ZTILE, _N - (_NS - 1) * _ZTILE)],
        )


def _node_body(h_ref, p_ref, w1a_ref, w1b_ref, b1_ref, w2_ref, b2_ref, o_ref):
    hb = h_ref[...]
    agg = p_ref[0] + p_ref[1]
    z = _silu(
        jnp.dot(hb, w1a_ref[...], preferred_element_type=jnp.float32)
        + jnp.dot(agg, w1b_ref[...], preferred_element_type=jnp.float32)
        + b1_ref[...]
    )
    o_ref[...] = (
        jnp.dot(z, w2_ref[...], preferred_element_type=jnp.float32) + b2_ref[...] + hb
    )


def _node_mlp(h, parts, w1a, w1b, b1, w2, b2):
    return pl.pallas_call(
        _node_body,
        grid=(_N // _BN,),
        in_specs=[
            pl.BlockSpec((_BN, _D), lambda n: (n, 0)),
            pl.BlockSpec((2, _BN, _H), lambda n: (0, n, 0)),
            pl.BlockSpec((_D, _H), lambda n: (0, 0)),
            pl.BlockSpec((_H, _H), lambda n: (0, 0)),
            pl.BlockSpec((1, _H), lambda n: (0, 0)),
            pl.BlockSpec((_H, _D), lambda n: (0, 0)),
            pl.BlockSpec((1, _D), lambda n: (0, 0)),
        ],
        out_specs=pl.BlockSpec((_BN, _D), lambda n: (n, 0)),
        out_shape=jax.ShapeDtypeStruct((_N, _D), jnp.float32),
    )(h, parts, w1a, w1b, b1, w2, b2)


def kernel(h, edge_index, edge_attr, eW1, eb1, eW2, eb2, nW1, nb1, nW2, nb2):
    row = edge_index[0]
    col = edge_index[1]
    w1a = eW1[:_D]
    w1b = eW1[_D : 2 * _D]
    w1c = eW1[2 * _D :]

    t = _premix(h, w1a, w1b)
    g = _sc_gather(t[0], t[1], row, col)
    f = _edge_mlp(
        g, edge_attr, w1c, eb1.reshape(1, _H), eW2, eb2.reshape(1, _H)
    )
    parts = _sc_scatter(f, row)
    out = _node_mlp(
        h,
        parts,
        nW1[:_D],
        nW1[_D:],
        nb1.reshape(1, _H),
        nW2,
        nb2.reshape(1, _D),
    )
    return (out, edge_attr)


# speedup vs baseline: 4.0576x; 1.0370x over previous
import functools

import jax
import jax.numpy as jnp
from jax import lax
from jax.experimental import pallas as pl
from jax.experimental.pallas import tpu as pltpu
from jax.experimental.pallas import tpu_sc as plsc

_N = 10000
_E = 320000
_D = 128
_DE = 16
_H = 128

_NC = 2
_NS = 16
_NW = _NC * _NS

_PER_W = _E // _NW
_CHUNK = 400
_NCHUNK = _PER_W // _CHUNK
_BATCH = 80
_KB = _CHUNK // _BATCH
_ACC_PAD = 10240
_ZTILE = _ACC_PAD // _NS
_NBATCH = _PER_W // _BATCH

_BN = 2000


def _silu(x):
    return x / (1.0 + jnp.exp(-x))


def _premix_body(h_ref, wa_ref, wb_ref, t_ref):
    hb = h_ref[...]
    t_ref[0] = jnp.dot(hb, wa_ref[...], preferred_element_type=jnp.float32)
    t_ref[1] = jnp.dot(hb, wb_ref[...], preferred_element_type=jnp.float32)


def _premix(h, wa, wb):
    return pl.pallas_call(
        _premix_body,
        grid=(_N // _BN,),
        in_specs=[
            pl.BlockSpec((_BN, _D), lambda n: (n, 0)),
            pl.BlockSpec((_D, _H), lambda n: (0, 0)),
            pl.BlockSpec((_D, _H), lambda n: (0, 0)),
        ],
        out_specs=pl.BlockSpec((2, _BN, _H), lambda n: (0, n, 0)),
        out_shape=jax.ShapeDtypeStruct((2, _N, _H), jnp.float32),
    )(h, wa, wb)


_sc_mesh = plsc.VectorSubcoreMesh(
    core_axis_name="c", subcore_axis_name="s", num_cores=_NC, num_subcores=_NS
)


@functools.partial(
    pl.kernel,
    out_type=jax.ShapeDtypeStruct((2, _E, _H), jnp.float32),
    mesh=_sc_mesh,
    scratch_types=[
        pltpu.VMEM((_PER_W,), jnp.int32),
        pltpu.VMEM((_PER_W,), jnp.int32),
        pltpu.VMEM((_CHUNK, _H), jnp.float32),
        pltpu.VMEM((_CHUNK, _H), jnp.float32),
        pltpu.SemaphoreType.DMA,
        pltpu.SemaphoreType.DMA,
        pltpu.SemaphoreType.DMA,
        pltpu.SemaphoreType.DMA,
    ],
)
def _sc_gather(t0_hbm, t1_hbm, row_hbm, col_hbm, g_hbm,
               idx_a, idx_b, rows_a, rows_b, sga, sgb, ssa, ssb):
    c = lax.axis_index("c")
    s = lax.axis_index("s")
    wid = c * _NS + s
    base = wid * _PER_W
    pltpu.sync_copy(row_hbm.at[pl.ds(base, _PER_W)], idx_a)
    pltpu.sync_copy(col_hbm.at[pl.ds(base, _PER_W)], idx_b)

    def store(half, rows_v, sem, ci):
        return pltpu.make_async_copy(
            rows_v, g_hbm.at[half, pl.ds(base + ci * _CHUNK, _CHUNK)], sem
        )

    def chunk_body(ci, carry):
        @pl.when(ci > 0)
        def _():
            store(0, rows_a, ssa, ci - 1).wait()
            store(1, rows_b, ssb, ci - 1).wait()

        cps = []
        for idx_v, rows_v, tab, sem in (
            (idx_a, rows_a, t0_hbm, sga),
            (idx_b, rows_b, t1_hbm, sgb),
        ):
            for j in range(_KB):
                sl = pl.ds(j * _BATCH, _BATCH)
                cps.append(
                    pltpu.async_copy(
                        tab.at[idx_v.at[pl.ds(ci * _CHUNK + j * _BATCH, _BATCH)]],
                        rows_v.at[sl],
                        sem,
                    )
                )
        for cp in cps[: _KB]:
            cp.wait()
        store(0, rows_a, ssa, ci).start()
        for cp in cps[_KB:]:
            cp.wait()
        store(1, rows_b, ssb, ci).start()
        return carry

    lax.fori_loop(0, _NCHUNK, chunk_body, 0)
    store(0, rows_a, ssa, _NCHUNK - 1).wait()
    store(1, rows_b, ssb, _NCHUNK - 1).wait()


def _edge_body(g_ref, attr_ref, w1c_ref, b1_ref, w2_ref, b2_ref, f_ref):
    z = (
        g_ref[0]
        + g_ref[1]
        + jnp.dot(attr_ref[...], w1c_ref[...], preferred_element_type=jnp.float32)
        + b1_ref[...]
    )
    z = _silu(z)
    f = jnp.dot(z, w2_ref[...], preferred_element_type=jnp.float32) + b2_ref[...]
    f_ref[...] = _silu(f)


def _edge_mlp(g, attr, w1c, b1, w2, b2):
    return pl.pallas_call(
        _edge_body,
        grid=(_E // _BN,),
        in_specs=[
            pl.BlockSpec((2, _BN, _H), lambda n: (0, n, 0)),
            pl.BlockSpec((_BN, _DE), lambda n: (n, 0)),
            pl.BlockSpec((_DE, _H), lambda n: (0, 0)),
            pl.BlockSpec((1, _H), lambda n: (0, 0)),
            pl.BlockSpec((_H, _H), lambda n: (0, 0)),
            pl.BlockSpec((1, _H), lambda n: (0, 0)),
        ],
        out_specs=pl.BlockSpec((_BN, _H), lambda n: (n, 0)),
        out_shape=jax.ShapeDtypeStruct((_E, _H), jnp.float32),
    )(g, attr, w1c, b1, w2, b2)


@functools.partial(
    pl.kernel,
    out_type=jax.ShapeDtypeStruct((_NC, _N, _H), jnp.float32),
    mesh=_sc_mesh,
    scratch_types=[
        pltpu.VMEM((_BATCH,), jnp.int32),
        pltpu.VMEM((_BATCH,), jnp.int32),
        pltpu.VMEM((_BATCH, _H), jnp.float32),
        pltpu.VMEM((_BATCH, _H), jnp.float32),
        pltpu.VMEM_SHARED((_ACC_PAD, _H), jnp.float32),
        pltpu.SemaphoreType.DMA,
        pltpu.SemaphoreType.DMA,
        pltpu.SemaphoreType.DMA,
        pltpu.SemaphoreType.DMA,
    ],
)
def _sc_scatter(f_hbm, row_hbm, out_hbm, i0, i1, f0, f1, acc, si0, si1, sf0, sf1):
    c = lax.axis_index("c")
    s = lax.axis_index("s")
    wid = c * _NS + s
    base = wid * _PER_W

    def zero_row(i, carry):
        for j in range(_H // 16):
            f0[i, pl.ds(j * 16, 16)] = jnp.zeros((16,), jnp.float32)
        return carry

    lax.fori_loop(0, _BATCH, zero_row, 0)
    for t in range(_ZTILE // _BATCH):
        pltpu.sync_copy(f0, acc.at[pl.ds(s * _ZTILE + t * _BATCH, _BATCH)])
    plsc.subcore_barrier()

    def start(e, iv, fvv, sem_i, sem_f):
        eb = base + e * _BATCH
        pltpu.async_copy(row_hbm.at[pl.ds(eb, _BATCH)], iv, sem_i)
        pltpu.async_copy(f_hbm.at[pl.ds(eb, _BATCH)], fvv, sem_f)

    def drain(e, iv, fvv, sem_i, sem_f):
        eb = base + e * _BATCH
        pltpu.make_async_copy(row_hbm.at[pl.ds(eb, _BATCH)], iv, sem_i).wait()
        pltpu.make_async_copy(f_hbm.at[pl.ds(eb, _BATCH)], fvv, sem_f).wait()

    start(0, i0, f0, si0, sf0)
    start(1, i1, f1, si1, sf1)

    def body(i, carry):
        e = 2 * i
        drain(e, i0, f0, si0, sf0)
        pltpu.sync_copy(f0, acc.at[i0], add=True)
        start(e + 2, i0, f0, si0, sf0)
        drain(e + 1, i1, f1, si1, sf1)
        pltpu.sync_copy(f1, acc.at[i1], add=True)

        @pl.when(i < _NBATCH // 2 - 1)
        def _():
            start(e + 3, i1, f1, si1, sf1)

        return carry

    lax.fori_loop(0, _NBATCH // 2, body, 0)
    drain(_NBATCH - 1, i0, f0, si0, sf0)
    pltpu.sync_copy(f0, acc.at[i0], add=True)
    plsc.subcore_barrier()
    @pl.when(s < _NS - 1)
    def _():
        pltpu.sync_copy(
            acc.at[pl.ds(s * _ZTILE, _ZTILE)],
            out_hbm.at[c, pl.ds(s * _ZTILE, _ZTILE)],
        )

    @pl.when(s == _NS - 1)
    def _():
        pltpu.sync_copy(
            acc.at[pl.ds((_NS - 1) * _ZTILE, _N - (_NS - 1) * _ZTILE)],
            out_hbm.at[c, pl.ds((_NS - 1) * _ZTILE, _N - (_NS - 1) * _ZTILE)],
        )


def _node_body(h_ref, p_ref, w1a_ref, w1b_ref, b1_ref, w2_ref, b2_ref, o_ref):
    hb = h_ref[...]
    agg = p_ref[0] + p_ref[1]
    z = _silu(
        jnp.dot(hb, w1a_ref[...], preferred_element_type=jnp.float32)
        + jnp.dot(agg, w1b_ref[...], preferred_element_type=jnp.float32)
        + b1_ref[...]
    )
    o_ref[...] = (
        jnp.dot(z, w2_ref[...], preferred_element_type=jnp.float32) + b2_ref[...] + hb
    )


def _node_mlp(h, parts, w1a, w1b, b1, w2, b2):
    return pl.pallas_call(
        _node_body,
        grid=(_N // _BN,),
        in_specs=[
            pl.BlockSpec((_BN, _D), lambda n: (n, 0)),
            pl.BlockSpec((2, _BN, _H), lambda n: (0, n, 0)),
            pl.BlockSpec((_D, _H), lambda n: (0, 0)),
            pl.BlockSpec((_H, _H), lambda n: (0, 0)),
            pl.BlockSpec((1, _H), lambda n: (0, 0)),
            pl.BlockSpec((_H, _D), lambda n: (0, 0)),
            pl.BlockSpec((1, _D), lambda n: (0, 0)),
        ],
        out_specs=pl.BlockSpec((_BN, _D), lambda n: (n, 0)),
        out_shape=jax.ShapeDtypeStruct((_N, _D), jnp.float32),
    )(h, parts, w1a, w1b, b1, w2, b2)


def kernel(h, edge_index, edge_attr, eW1, eb1, eW2, eb2, nW1, nb1, nW2, nb2):
    row = edge_index[0]
    col = edge_index[1]
    w1a = eW1[:_D]
    w1b = eW1[_D : 2 * _D]
    w1c = eW1[2 * _D :]

    t = _premix(h, w1a, w1b)
    g = _sc_gather(t[0], t[1], row, col)
    f = _edge_mlp(
        g, edge_attr, w1c, eb1.reshape(1, _H), eW2, eb2.reshape(1, _H)
    )
    parts = _sc_scatter(f, row)
    out = _node_mlp(
        h,
        parts,
        nW1[:_D],
        nW1[_D:],
        nb1.reshape(1, _H),
        nW2,
        nb2.reshape(1, _D),
    )
    return (out, edge_attr)


# speedup vs baseline: 4.4372x; 1.0936x over previous
import functools

import jax
import jax.numpy as jnp
from jax import lax
from jax.experimental import pallas as pl
from jax.experimental.pallas import tpu as pltpu
from jax.experimental.pallas import tpu_sc as plsc

_N = 10000
_E = 320000
_D = 128
_DE = 16
_H = 128

_NC = 2
_NS = 16
_NW = _NC * _NS

_PER_W = _E // _NW
_CHUNK = 400
_NCHUNK = _PER_W // _CHUNK
_BATCH = 80
_KB = _CHUNK // _BATCH
_ACC_PAD = 10240
_ZTILE = _ACC_PAD // _NS
_NBATCH = _PER_W // _BATCH

_BN = 2000


def _silu(x):
    return x / (1.0 + jnp.exp(-x))


def _premix_body(h_ref, wa_ref, wb_ref, t_ref):
    hb = h_ref[...]
    t_ref[0] = jnp.dot(hb, wa_ref[...], preferred_element_type=jnp.float32)
    t_ref[1] = jnp.dot(hb, wb_ref[...], preferred_element_type=jnp.float32)


def _premix(h, wa, wb):
    return pl.pallas_call(
        _premix_body,
        grid=(_N // _BN,),
        in_specs=[
            pl.BlockSpec((_BN, _D), lambda n: (n, 0)),
            pl.BlockSpec((_D, _H), lambda n: (0, 0)),
            pl.BlockSpec((_D, _H), lambda n: (0, 0)),
        ],
        out_specs=pl.BlockSpec((2, _BN, _H), lambda n: (0, n, 0)),
        out_shape=jax.ShapeDtypeStruct((2, _N, _H), jnp.float32),
    )(h, wa, wb)


_sc_mesh = plsc.VectorSubcoreMesh(
    core_axis_name="c", subcore_axis_name="s", num_cores=_NC, num_subcores=_NS
)


@functools.partial(
    pl.kernel,
    out_type=jax.ShapeDtypeStruct((_E, _H), jnp.float32),
    mesh=_sc_mesh,
    scratch_types=[
        pltpu.VMEM((_PER_W,), jnp.int32),
        pltpu.VMEM((_PER_W,), jnp.int32),
        pltpu.VMEM((_CHUNK, _H), jnp.float32),
        pltpu.VMEM((_CHUNK, _H), jnp.float32),
        pltpu.SemaphoreType.DMA,
        pltpu.SemaphoreType.DMA,
        pltpu.SemaphoreType.DMA,
        pltpu.SemaphoreType.DMA,
    ],
)
def _sc_gather(t0_hbm, t1_hbm, row_hbm, col_hbm, g_hbm,
               idx_a, idx_b, rows_a, rows_b, sga, sgb, ssa, ssb):
    c = lax.axis_index("c")
    s = lax.axis_index("s")
    wid = c * _NS + s
    base = wid * _PER_W
    pltpu.sync_copy(row_hbm.at[pl.ds(base, _PER_W)], idx_a)
    pltpu.sync_copy(col_hbm.at[pl.ds(base, _PER_W)], idx_b)

    def store(ci):
        return pltpu.make_async_copy(
            rows_a, g_hbm.at[pl.ds(base + ci * _CHUNK, _CHUNK)], ssa
        )

    def fire(idx_v, rows_v, tab, sem, ci):
        return [
            pltpu.async_copy(
                tab.at[idx_v.at[pl.ds(ci * _CHUNK + j * _BATCH, _BATCH)]],
                rows_v.at[pl.ds(j * _BATCH, _BATCH)],
                sem,
            )
            for j in range(_KB)
        ]

    def chunk_body(ci, carry):
        cps_b = fire(idx_b, rows_b, t1_hbm, sgb, ci)

        @pl.when(ci > 0)
        def _():
            store(ci - 1).wait()

        cps_a = fire(idx_a, rows_a, t0_hbm, sga, ci)
        for cp in cps_a + cps_b:
            cp.wait()

        def add_row(r, carry2):
            for j in range(_H // 16):
                sl = pl.ds(j * 16, 16)
                rows_a[r, sl] = rows_a[r, sl] + rows_b[r, sl]
            return carry2

        lax.fori_loop(0, _CHUNK, add_row, 0)
        store(ci).start()
        return carry

    lax.fori_loop(0, _NCHUNK, chunk_body, 0)
    store(_NCHUNK - 1).wait()


def _edge_body(g_ref, attr_ref, w1c_ref, b1_ref, w2_ref, b2_ref, f_ref):
    z = (
        g_ref[...]
        + jnp.dot(attr_ref[...], w1c_ref[...], preferred_element_type=jnp.float32)
        + b1_ref[...]
    )
    z = _silu(z)
    f = jnp.dot(z, w2_ref[...], preferred_element_type=jnp.float32) + b2_ref[...]
    f_ref[...] = _silu(f)


def _edge_mlp(g, attr, w1c, b1, w2, b2):
    return pl.pallas_call(
        _edge_body,
        grid=(_E // _BN,),
        in_specs=[
            pl.BlockSpec((_BN, _H), lambda n: (n, 0)),
            pl.BlockSpec((_BN, _DE), lambda n: (n, 0)),
            pl.BlockSpec((_DE, _H), lambda n: (0, 0)),
            pl.BlockSpec((1, _H), lambda n: (0, 0)),
            pl.BlockSpec((_H, _H), lambda n: (0, 0)),
            pl.BlockSpec((1, _H), lambda n: (0, 0)),
        ],
        out_specs=pl.BlockSpec((_BN, _H), lambda n: (n, 0)),
        out_shape=jax.ShapeDtypeStruct((_E, _H), jnp.float32),
    )(g, attr, w1c, b1, w2, b2)


@functools.partial(
    pl.kernel,
    out_type=jax.ShapeDtypeStruct((_NC, _N, _H), jnp.float32),
    mesh=_sc_mesh,
    scratch_types=[
        pltpu.VMEM((_BATCH,), jnp.int32),
        pltpu.VMEM((_BATCH,), jnp.int32),
        pltpu.VMEM((_BATCH, _H), jnp.float32),
        pltpu.VMEM((_BATCH, _H), jnp.float32),
        pltpu.VMEM_SHARED((_ACC_PAD, _H), jnp.float32),
        pltpu.SemaphoreType.DMA,
        pltpu.SemaphoreType.DMA,
        pltpu.SemaphoreType.DMA,
        pltpu.SemaphoreType.DMA,
    ],
)
def _sc_scatter(f_hbm, row_hbm, out_hbm, i0, i1, f0, f1, acc, si0, si1, sf0, sf1):
    c = lax.axis_index("c")
    s = lax.axis_index("s")
    wid = c * _NS + s
    base = wid * _PER_W

    def zero_row(i, carry):
        for j in range(_H // 16):
            f0[i, pl.ds(j * 16, 16)] = jnp.zeros((16,), jnp.float32)
        return carry

    lax.fori_loop(0, _BATCH, zero_row, 0)
    for t in range(_ZTILE // _BATCH):
        pltpu.sync_copy(f0, acc.at[pl.ds(s * _ZTILE + t * _BATCH, _BATCH)])
    plsc.subcore_barrier()

    def start(e, iv, fvv, sem_i, sem_f):
        eb = base + e * _BATCH
        pltpu.async_copy(row_hbm.at[pl.ds(eb, _BATCH)], iv, sem_i)
        pltpu.async_copy(f_hbm.at[pl.ds(eb, _BATCH)], fvv, sem_f)

    def drain(e, iv, fvv, sem_i, sem_f):
        eb = base + e * _BATCH
        pltpu.make_async_copy(row_hbm.at[pl.ds(eb, _BATCH)], iv, sem_i).wait()
        pltpu.make_async_copy(f_hbm.at[pl.ds(eb, _BATCH)], fvv, sem_f).wait()

    start(0, i0, f0, si0, sf0)
    start(1, i1, f1, si1, sf1)

    def body(i, carry):
        e = 2 * i
        drain(e, i0, f0, si0, sf0)
        pltpu.sync_copy(f0, acc.at[i0], add=True)
        start(e + 2, i0, f0, si0, sf0)
        drain(e + 1, i1, f1, si1, sf1)
        pltpu.sync_copy(f1, acc.at[i1], add=True)

        @pl.when(i < _NBATCH // 2 - 1)
        def _():
            start(e + 3, i1, f1, si1, sf1)

        return carry

    lax.fori_loop(0, _NBATCH // 2, body, 0)
    drain(_NBATCH - 1, i0, f0, si0, sf0)
    pltpu.sync_copy(f0, acc.at[i0], add=True)
    plsc.subcore_barrier()
    @pl.when(s < _NS - 1)
    def _():
        pltpu.sync_copy(
            acc.at[pl.ds(s * _ZTILE, _ZTILE)],
            out_hbm.at[c, pl.ds(s * _ZTILE, _ZTILE)],
        )

    @pl.when(s == _NS - 1)
    def _():
        pltpu.sync_copy(
            acc.at[pl.ds((_NS - 1) * _ZTILE, _N - (_NS - 1) * _ZTILE)],
            out_hbm.at[c, pl.ds((_NS - 1) * _ZTILE, _N - (_NS - 1) * _ZTILE)],
        )


def _node_body(h_ref, p_ref, w1a_ref, w1b_ref, b1_ref, w2_ref, b2_ref, o_ref):
    hb = h_ref[...]
    agg = p_ref[0] + p_ref[1]
    z = _silu(
        jnp.dot(hb, w1a_ref[...], preferred_element_type=jnp.float32)
        + jnp.dot(agg, w1b_ref[...], preferred_element_type=jnp.float32)
        + b1_ref[...]
    )
    o_ref[...] = (
        jnp.dot(z, w2_ref[...], preferred_element_type=jnp.float32) + b2_ref[...] + hb
    )


def _node_mlp(h, parts, w1a, w1b, b1, w2, b2):
    return pl.pallas_call(
        _node_body,
        grid=(_N // _BN,),
        in_specs=[
            pl.BlockSpec((_BN, _D), lambda n: (n, 0)),
            pl.BlockSpec((2, _BN, _H), lambda n: (0, n, 0)),
            pl.BlockSpec((_D, _H), lambda n: (0, 0)),
            pl.BlockSpec((_H, _H), lambda n: (0, 0)),
            pl.BlockSpec((1, _H), lambda n: (0, 0)),
            pl.BlockSpec((_H, _D), lambda n: (0, 0)),
            pl.BlockSpec((1, _D), lambda n: (0, 0)),
        ],
        out_specs=pl.BlockSpec((_BN, _D), lambda n: (n, 0)),
        out_shape=jax.ShapeDtypeStruct((_N, _D), jnp.float32),
    )(h, parts, w1a, w1b, b1, w2, b2)


def kernel(h, edge_index, edge_attr, eW1, eb1, eW2, eb2, nW1, nb1, nW2, nb2):
    row = edge_index[0]
    col = edge_index[1]
    w1a = eW1[:_D]
    w1b = eW1[_D : 2 * _D]
    w1c = eW1[2 * _D :]

    t = _premix(h, w1a, w1b)
    g = _sc_gather(t[0], t[1], row, col)
    f = _edge_mlp(
        g, edge_attr, w1c, eb1.reshape(1, _H), eW2, eb2.reshape(1, _H)
    )
    parts = _sc_scatter(f, row)
    out = _node_mlp(
        h,
        parts,
        nW1[:_D],
        nW1[_D:],
        nb1.reshape(1, _H),
        nW2,
        nb2.reshape(1, _D),
    )
    return (out, edge_attr)
